# baseline (speedup 1.0000x reference)
import functools

import jax
import jax.numpy as jnp
from jax import lax
from jax.experimental import pallas as pl
from jax.experimental.pallas import tpu as pltpu
from jax.experimental.pallas import tpu_sc as plsc

N = 10000
E = 320000
D = 128
B = 125
NTILE = 16
NW = 32
NBLK = E // B
BPT = NBLK // NW
HALF = BPT // 2
NPAD = 10240
RPT = NPAD // NTILE

_MESH = plsc.VectorSubcoreMesh(core_axis_name="c", subcore_axis_name="s")
_f32 = jnp.float32


@functools.partial(
    pl.kernel,
    mesh=_MESH,
    out_type=jax.ShapeDtypeStruct((2, NPAD), _f32),
    scratch_types=[
        pltpu.VMEM((BPT, B), jnp.int32),
        pltpu.VMEM((B,), _f32),
        pltpu.VMEM_SHARED((NPAD,), _f32),
    ],
)
def _sc_degree(dst2_hbm, zeros1_hbm, ones_hbm, out_hbm, dsts_v, ones_v, acc_sh):
    c = lax.axis_index("c")
    s = lax.axis_index("s")
    w = c * NTILE + s
    pltpu.sync_copy(zeros1_hbm, acc_sh.at[pl.ds(s * RPT, RPT)])
    pltpu.sync_copy(ones_hbm, ones_v)
    pltpu.sync_copy(dst2_hbm.at[pl.ds(w * BPT, BPT), :], dsts_v)
    plsc.subcore_barrier()

    @pl.loop(0, BPT)
    def _(j):
        pltpu.sync_copy(ones_v, acc_sh.at[dsts_v.at[j]], add=True)

    plsc.subcore_barrier()
    pltpu.sync_copy(
        acc_sh.at[pl.ds(s * RPT, RPT)],
        out_hbm.at[c, pl.ds(s * RPT, RPT)],
    )


@functools.partial(
    pl.kernel,
    mesh=_MESH,
    out_type=[
        jax.ShapeDtypeStruct((NPAD, D), _f32),
        jax.ShapeDtypeStruct((NPAD, D), _f32),
    ],
    scratch_types=[
        pltpu.VMEM((HALF, B), jnp.int32),
        pltpu.VMEM((HALF, B), jnp.int32),
        pltpu.VMEM((B, D), _f32),
        pltpu.VMEM((B, D), _f32),
        pltpu.VMEM_SHARED((NPAD, D), _f32),
        pltpu.SemaphoreType.DMA,
        pltpu.SemaphoreType.DMA,
    ],
)
def _sc_prop(t_hbm, src2_hbm, dst2_hbm, zeros2_hbm, o0_hbm, o1_hbm,
             srcs_v, dsts_v, rows0, rows1, acc_sh, sem0, sem1):
    c = lax.axis_index("c")
    s = lax.axis_index("s")
    w = c * NTILE + s
    pltpu.sync_copy(zeros2_hbm, acc_sh.at[pl.ds(s * RPT, RPT), :])
    plsc.subcore_barrier()

    for half in range(2):
        base = w * BPT + half * HALF
        pltpu.sync_copy(src2_hbm.at[pl.ds(base, HALF), :], srcs_v)
        pltpu.sync_copy(dst2_hbm.at[pl.ds(base, HALF), :], dsts_v)
        pltpu.async_copy(t_hbm.at[srcs_v.at[0]], rows0, sem0)

        @pl.loop(0, HALF // 2)
        def _(k):
            j = 2 * k
            pltpu.async_copy(t_hbm.at[srcs_v.at[j + 1]], rows1, sem1)
            pltpu.make_async_copy(t_hbm.at[srcs_v.at[j]], rows0, sem0).wait()
            pltpu.sync_copy(rows0, acc_sh.at[dsts_v.at[j]], add=True)

            @pl.when(j + 2 < HALF)
            def _():
                pltpu.async_copy(t_hbm.at[srcs_v.at[j + 2]], rows0, sem0)

            pltpu.make_async_copy(t_hbm.at[srcs_v.at[j + 1]], rows1, sem1).wait()
            pltpu.sync_copy(rows1, acc_sh.at[dsts_v.at[j + 1]], add=True)

    plsc.subcore_barrier()
    tile_rows = pl.ds(s * RPT, RPT)

    @pl.when(c == 0)
    def _():
        pltpu.sync_copy(acc_sh.at[tile_rows, :], o0_hbm.at[tile_rows, :])

    @pl.when(c == 1)
    def _():
        pltpu.sync_copy(acc_sh.at[tile_rows, :], o1_hbm.at[tile_rows, :])


def _stage_matmul(feat_ref, w_ref, y0_ref):
    y0_ref[...] = jnp.dot(
        feat_ref[...], w_ref[...], preferred_element_type=jnp.float32)


def _stage_scale0(dp0_ref, dp1_ref, y0_ref, t0_ref, dinv_ref, dinv2_ref):
    deg = dp0_ref[...] + dp1_ref[...] + 1.0
    di = lax.rsqrt(deg)
    dinv_ref[...] = di
    dinv2_ref[...] = 1.0 / deg
    t0_ref[...] = y0_ref[...] * di


def _stage_mid(u0_ref, u1_ref, t0_ref, dinv2_ref, t1_ref):
    u = u0_ref[...] + u1_ref[...]
    t1_ref[...] = (u + t0_ref[...]) * dinv2_ref[...]


def _stage_final(u0_ref, u1_ref, t1_ref, dinv_ref, b_ref, out_ref):
    u = u0_ref[...] + u1_ref[...]
    out_ref[...] = (u + t1_ref[...]) * dinv_ref[...] + b_ref[...]


RB = N // 10


def _row_spec(cols):
    return pl.BlockSpec((RB, cols), lambda i: (i, 0))


@jax.jit
def kernel(feat, edge_index, W, b):
    src = edge_index[0].astype(jnp.int32).reshape(NBLK, B)
    dst = edge_index[1].astype(jnp.int32).reshape(NBLK, B)
    zeros1 = jnp.zeros((RPT,), _f32)
    zeros2 = jnp.zeros((RPT, D), _f32)
    ones = jnp.ones((B,), _f32)

    y0 = pl.pallas_call(
        _stage_matmul,
        out_shape=jax.ShapeDtypeStruct((N, D), _f32),
    )(feat, W)
    deg_p = _sc_degree(dst, zeros1, ones)
    dp0 = deg_p[0, :N].reshape(N, 1)
    dp1 = deg_p[1, :N].reshape(N, 1)

    t0, dinv, dinv2 = pl.pallas_call(
        _stage_scale0,
        grid=(10,),
        in_specs=[_row_spec(1), _row_spec(1), _row_spec(D)],
        out_specs=[_row_spec(D), _row_spec(1), _row_spec(1)],
        out_shape=[
            jax.ShapeDtypeStruct((N, D), _f32),
            jax.ShapeDtypeStruct((N, 1), _f32),
            jax.ShapeDtypeStruct((N, 1), _f32),
        ],
    )(dp0, dp1, y0)

    u10, u11 = _sc_prop(t0, src, dst, zeros2)
    t1 = pl.pallas_call(
        _stage_mid,
        grid=(10,),
        in_specs=[_row_spec(D), _row_spec(D), _row_spec(D), _row_spec(1)],
        out_specs=_row_spec(D),
        out_shape=jax.ShapeDtypeStruct((N, D), _f32),
    )(u10, u11, t0, dinv2)

    u20, u21 = _sc_prop(t1, src, dst, zeros2)
    out = pl.pallas_call(
        _stage_final,
        grid=(10,),
        in_specs=[_row_spec(D), _row_spec(D), _row_spec(D), _row_spec(1),
                  pl.BlockSpec((D,), lambda i: (0,))],
        out_specs=_row_spec(D),
        out_shape=jax.ShapeDtypeStruct((N, D), _f32),
    )(u20, u21, t1, dinv, b)
    return out


# speedup vs baseline: 1.0152x; 1.0152x over previous
import functools

import jax
import jax.numpy as jnp
from jax import lax
from jax.experimental import pallas as pl
from jax.experimental.pallas import tpu as pltpu
from jax.experimental.pallas import tpu_sc as plsc

N = 10000
E = 320000
D = 128
B = 125
NTILE = 16
NW = 32
NBLK = E // B
BPT = NBLK // NW
HALF = BPT // 2
NPAD = 10240
RPT = NPAD // NTILE

_MESH = plsc.VectorSubcoreMesh(core_axis_name="c", subcore_axis_name="s")
_f32 = jnp.float32


@functools.partial(
    pl.kernel,
    mesh=_MESH,
    out_type=jax.ShapeDtypeStruct((2, NPAD), _f32),
    scratch_types=[
        pltpu.VMEM((BPT, B), jnp.int32),
        pltpu.VMEM((B,), _f32),
        pltpu.VMEM_SHARED((NPAD,), _f32),
    ],
)
def _sc_degree(dst2_hbm, zeros1_hbm, ones_hbm, out_hbm, dsts_v, ones_v, acc_sh):
    c = lax.axis_index("c")
    s = lax.axis_index("s")
    w = c * NTILE + s
    pltpu.sync_copy(zeros1_hbm, acc_sh.at[pl.ds(s * RPT, RPT)])
    pltpu.sync_copy(ones_hbm, ones_v)
    pltpu.sync_copy(dst2_hbm.at[pl.ds(w * BPT, BPT), :], dsts_v)
    plsc.subcore_barrier()

    @pl.loop(0, BPT)
    def _(j):
        pltpu.sync_copy(ones_v, acc_sh.at[dsts_v.at[j]], add=True)

    plsc.subcore_barrier()
    pltpu.sync_copy(
        acc_sh.at[pl.ds(s * RPT, RPT)],
        out_hbm.at[c, pl.ds(s * RPT, RPT)],
    )


@functools.partial(
    pl.kernel,
    mesh=_MESH,
    out_type=[
        jax.ShapeDtypeStruct((NPAD, D), _f32),
        jax.ShapeDtypeStruct((NPAD, D), _f32),
    ],
    scratch_types=[
        pltpu.VMEM((HALF, B), jnp.int32),
        pltpu.VMEM((HALF, B), jnp.int32),
        pltpu.VMEM((B, D), _f32),
        pltpu.VMEM((B, D), _f32),
        pltpu.VMEM_SHARED((NPAD, D), _f32),
        pltpu.SemaphoreType.DMA,
        pltpu.SemaphoreType.DMA,
    ],
)
def _sc_prop(t_hbm, src2_hbm, dst2_hbm, zeros2_hbm, o0_hbm, o1_hbm,
             srcs_v, dsts_v, rows0, rows1, acc_sh, sem0, sem1):
    c = lax.axis_index("c")
    s = lax.axis_index("s")
    w = c * NTILE + s
    pltpu.sync_copy(zeros2_hbm, acc_sh.at[pl.ds(s * RPT, RPT), :])
    plsc.subcore_barrier()

    for half in range(2):
        base = w * BPT + half * HALF
        pltpu.sync_copy(src2_hbm.at[pl.ds(base, HALF), :], srcs_v)
        pltpu.sync_copy(dst2_hbm.at[pl.ds(base, HALF), :], dsts_v)
        pltpu.async_copy(t_hbm.at[srcs_v.at[0]], rows0, sem0)

        @pl.loop(0, HALF // 2)
        def _(k):
            j = 2 * k
            pltpu.async_copy(t_hbm.at[srcs_v.at[j + 1]], rows1, sem1)
            pltpu.make_async_copy(t_hbm.at[srcs_v.at[j]], rows0, sem0).wait()
            pltpu.sync_copy(rows0, acc_sh.at[dsts_v.at[j]], add=True)

            @pl.when(j + 2 < HALF)
            def _():
                pltpu.async_copy(t_hbm.at[srcs_v.at[j + 2]], rows0, sem0)

            pltpu.make_async_copy(t_hbm.at[srcs_v.at[j + 1]], rows1, sem1).wait()
            pltpu.sync_copy(rows1, acc_sh.at[dsts_v.at[j + 1]], add=True)

    plsc.subcore_barrier()
    tile_rows = pl.ds(s * RPT, RPT)

    @pl.when(c == 0)
    def _():
        pltpu.sync_copy(acc_sh.at[tile_rows, :], o0_hbm.at[tile_rows, :])

    @pl.when(c == 1)
    def _():
        pltpu.sync_copy(acc_sh.at[tile_rows, :], o1_hbm.at[tile_rows, :])


def _stage_matmul(feat_ref, w_ref, y0_ref):
    y0_ref[...] = jnp.dot(
        feat_ref[...], w_ref[...], preferred_element_type=jnp.float32)


def _stage_scale0(dp0_ref, dp1_ref, y0_ref, t0_ref, dinv_ref, dinv2_ref):
    deg = dp0_ref[...] + dp1_ref[...] + 1.0
    di = lax.rsqrt(deg)
    dinv_ref[...] = di
    dinv2_ref[...] = 1.0 / deg
    t0_ref[...] = y0_ref[...] * di


def _stage_mid(u0_ref, u1_ref, t0_ref, dinv2_ref, t1_ref):
    u = u0_ref[:N, :] + u1_ref[:N, :]
    t1_ref[...] = (u + t0_ref[...]) * dinv2_ref[...]


def _stage_final(u0_ref, u1_ref, t1_ref, dinv_ref, b_ref, out_ref):
    u = u0_ref[:N, :] + u1_ref[:N, :]
    out_ref[...] = (u + t1_ref[...]) * dinv_ref[...] + b_ref[...]


@jax.jit
def kernel(feat, edge_index, W, b):
    src = edge_index[0].astype(jnp.int32).reshape(NBLK, B)
    dst = edge_index[1].astype(jnp.int32).reshape(NBLK, B)
    zeros1 = jnp.zeros((RPT,), _f32)
    zeros2 = jnp.zeros((RPT, D), _f32)
    ones = jnp.ones((B,), _f32)

    y0 = pl.pallas_call(
        _stage_matmul,
        out_shape=jax.ShapeDtypeStruct((N, D), _f32),
    )(feat, W)
    deg_p = _sc_degree(dst, zeros1, ones)
    dp0 = deg_p[0, :N].reshape(N, 1)
    dp1 = deg_p[1, :N].reshape(N, 1)

    t0, dinv, dinv2 = pl.pallas_call(
        _stage_scale0,
        out_shape=[
            jax.ShapeDtypeStruct((N, D), _f32),
            jax.ShapeDtypeStruct((N, 1), _f32),
            jax.ShapeDtypeStruct((N, 1), _f32),
        ],
    )(dp0, dp1, y0)

    u10, u11 = _sc_prop(t0, src, dst, zeros2)
    t1 = pl.pallas_call(
        _stage_mid,
        out_shape=jax.ShapeDtypeStruct((N, D), _f32),
    )(u10, u11, t0, dinv2)

    u20, u21 = _sc_prop(t1, src, dst, zeros2)
    out = pl.pallas_call(
        _stage_final,
        out_shape=jax.ShapeDtypeStruct((N, D), _f32),
    )(u20, u21, t1, dinv, b)
    return out
